# baseline (speedup 1.0000x reference)
import functools

import jax
import jax.numpy as jnp
from jax import lax
from jax.experimental import pallas as pl
from jax.experimental.pallas import tpu as pltpu
from jax.experimental.pallas import tpu_sc as plsc

_NC = 2
_NS = 16
_CHUNK = 32


def kernel(tokens, W_pos):
    batch, seq = tokens.shape
    d_model = W_pos.shape[-1]
    nw = _NC * _NS
    rows_per_w = seq // nw
    n_chunks = rows_per_w // _CHUNK
    mesh = plsc.VectorSubcoreMesh(core_axis_name="c", subcore_axis_name="s",
                                  num_cores=_NC, num_subcores=_NS)

    @functools.partial(
        pl.kernel, mesh=mesh,
        out_type=jax.ShapeDtypeStruct((batch, seq, d_model), W_pos.dtype),
        scratch_types=[
            pltpu.VMEM((2, _CHUNK, d_model), W_pos.dtype),
            pltpu.SemaphoreType.DMA((2,)),
            pltpu.SemaphoreType.DMA,
        ],
    )
    def sc_body(w_hbm, o_hbm, buf, in_sems, out_sem):
        wid = lax.axis_index("s") * _NC + lax.axis_index("c")
        base = wid * rows_per_w
        cins = []
        for k in range(min(2, n_chunks)):
            rows = pl.ds(base + k * _CHUNK, _CHUNK)
            cp = pltpu.make_async_copy(w_hbm.at[rows, :], buf.at[k % 2],
                                       in_sems.at[k % 2])
            cp.start()
            cins.append(cp)
        outs = []
        for k in range(n_chunks):
            slot = k % 2
            cins[k].wait()
            rows = pl.ds(base + k * _CHUNK, _CHUNK)
            wouts = []
            for b in range(batch):
                cp = pltpu.make_async_copy(buf.at[slot],
                                           o_hbm.at[b, rows, :], out_sem)
                cp.start()
                wouts.append(cp)
            outs.append(wouts)
            if k + 2 < n_chunks:
                for cp2 in outs[k]:
                    cp2.wait()
                rows2 = pl.ds(base + (k + 2) * _CHUNK, _CHUNK)
                cp = pltpu.make_async_copy(w_hbm.at[rows2, :], buf.at[slot],
                                           in_sems.at[slot])
                cp.start()
                cins.append(cp)
        for k in range(max(0, n_chunks - 2), n_chunks):
            for cp2 in outs[k]:
                cp2.wait()

    return sc_body(W_pos)


# speedup vs baseline: 1.7789x; 1.7789x over previous
import jax
import jax.numpy as jnp
from jax.experimental import pallas as pl
from jax.experimental.pallas import tpu as pltpu

_N_CHUNKS = 8
_READ_AHEAD = 2


def _body(w_hbm, o_hbm, vbuf, in_sems, out_sem):
    seq = vbuf.shape[0]
    batch = o_hbm.shape[0]
    blk = seq // _N_CHUNKS

    def in_cp(c):
        rows = pl.ds(c * blk, blk)
        return pltpu.make_async_copy(w_hbm.at[rows, :], vbuf.at[rows, :],
                                     in_sems.at[c])

    for c in range(min(_READ_AHEAD, _N_CHUNKS)):
        in_cp(c).start()
    couts = []
    for c in range(_N_CHUNKS):
        in_cp(c).wait()
        if c + _READ_AHEAD < _N_CHUNKS:
            in_cp(c + _READ_AHEAD).start()
        rows = pl.ds(c * blk, blk)
        for b in range(batch):
            cp = pltpu.make_async_copy(vbuf.at[rows, :],
                                       o_hbm.at[b, rows, :], out_sem)
            cp.start()
            couts.append(cp)
    for cp in couts:
        cp.wait()


def kernel(tokens, W_pos):
    batch, seq = tokens.shape
    d_model = W_pos.shape[-1]
    return pl.pallas_call(
        _body,
        in_specs=[pl.BlockSpec(memory_space=pl.ANY)],
        out_specs=pl.BlockSpec(memory_space=pl.ANY),
        out_shape=jax.ShapeDtypeStruct((batch, seq, d_model), W_pos.dtype),
        scratch_shapes=[
            pltpu.VMEM((seq, d_model), W_pos.dtype),
            pltpu.SemaphoreType.DMA((_N_CHUNKS,)),
            pltpu.SemaphoreType.DMA,
        ],
    )(W_pos)


# speedup vs baseline: 1.8714x; 1.0520x over previous
import jax
import jax.numpy as jnp
from jax.experimental import pallas as pl
from jax.experimental.pallas import tpu as pltpu

_N_CHUNKS = 4
_OUT_SPLIT = 2


def _body(w_hbm, o_hbm, vbuf, in_sems, out_sem):
    seq = vbuf.shape[0]
    batch = o_hbm.shape[0]
    blk = seq // _N_CHUNKS
    oblk = blk // _OUT_SPLIT

    cins = []
    for c in range(_N_CHUNKS):
        rows = pl.ds(c * blk, blk)
        cp = pltpu.make_async_copy(w_hbm.at[rows, :], vbuf.at[rows, :],
                                   in_sems.at[c])
        cp.start()
        cins.append(cp)
    couts = []
    for c in range(_N_CHUNKS):
        cins[c].wait()
        for s in range(_OUT_SPLIT):
            rows = pl.ds(c * blk + s * oblk, oblk)
            for b in range(batch):
                cp = pltpu.make_async_copy(vbuf.at[rows, :],
                                           o_hbm.at[b, rows, :], out_sem)
                cp.start()
                couts.append(cp)
    for cp in couts:
        cp.wait()


def kernel(tokens, W_pos):
    batch, seq = tokens.shape
    d_model = W_pos.shape[-1]
    return pl.pallas_call(
        _body,
        in_specs=[pl.BlockSpec(memory_space=pl.ANY)],
        out_specs=pl.BlockSpec(memory_space=pl.ANY),
        out_shape=jax.ShapeDtypeStruct((batch, seq, d_model), W_pos.dtype),
        scratch_shapes=[
            pltpu.VMEM((seq, d_model), W_pos.dtype),
            pltpu.SemaphoreType.DMA((_N_CHUNKS,)),
            pltpu.SemaphoreType.DMA,
        ],
    )(W_pos)
